# baseline (speedup 1.0000x reference)
import jax
import jax.numpy as jnp
from jax import lax
from jax.experimental import pallas as pl
from jax.experimental.pallas import tpu as pltpu
from jax.experimental.pallas import tpu_sc as plsc

B = 16384
D = 64
NC = 2
NS = 16
L = 16
NW = NC * NS
BPW = B // NW
C = 128
NCH = BPW // C


def _sc_body(u_idx, p_idx, n_idx, utab, itab, out_pos, out_neg,
             idx_u, idx_p, idx_n, rows, acc_pos_v, acc_neg_v, sems):
    wid = lax.axis_index("s") * NC + lax.axis_index("c")
    base = wid * BPW

    for j in range(NCH):
        sl = pl.ds(base + j * C, C)
        pltpu.sync_copy(u_idx.at[sl], idx_u.at[j])
        pltpu.sync_copy(p_idx.at[sl], idx_p.at[j])
        pltpu.sync_copy(n_idx.at[sl], idx_n.at[j])

    def chunk_body(c, _):
        handles = []
        for g in range(C // L):
            sl = pl.ds(g * L, L)
            vu = idx_u[c, sl]
            vp = idx_p[c, sl]
            vn = idx_n[c, sl]
            for j in range(L):
                e = g * L + j
                handles.append(pltpu.async_copy(
                    utab.at[pl.ds(vu[j], 1)], rows.at[0, pl.ds(e, 1)],
                    sems.at[0]))
                handles.append(pltpu.async_copy(
                    itab.at[pl.ds(vp[j], 1)], rows.at[1, pl.ds(e, 1)],
                    sems.at[1]))
                handles.append(pltpu.async_copy(
                    itab.at[pl.ds(vn[j], 1)], rows.at[2, pl.ds(e, 1)],
                    sems.at[2]))
        for h in handles:
            h.wait()

        for g in range(C // L):
            ev = lax.iota(jnp.int32, L) + g * L

            def col_body(_, carry):
                accp, accn, colv = carry
                vu = plsc.load_gather(rows.at[0], [ev, colv])
                vp = plsc.load_gather(rows.at[1], [ev, colv])
                vn = plsc.load_gather(rows.at[2], [ev, colv])
                return (accp + vu * vp, accn + vu * vn, colv + 1)

            accp, accn, _ = lax.fori_loop(
                0, D, col_body,
                (jnp.zeros((L,), jnp.float32), jnp.zeros((L,), jnp.float32),
                 jnp.zeros((L,), jnp.int32)))
            acc_pos_v[pl.ds(c * C + g * L, L)] = accp
            acc_neg_v[pl.ds(c * C + g * L, L)] = accn
        return 0

    lax.fori_loop(0, NCH, chunk_body, 0)

    pltpu.sync_copy(acc_pos_v, out_pos.at[pl.ds(base, BPW)])
    pltpu.sync_copy(acc_neg_v, out_neg.at[pl.ds(base, BPW)])


@jax.jit
def kernel(user_inputs, pos_item_inputs, neg_item_inputs, user_table, item_table):
    mesh = plsc.VectorSubcoreMesh(core_axis_name="c", subcore_axis_name="s")
    f = pl.kernel(
        _sc_body,
        out_type=(jax.ShapeDtypeStruct((B,), jnp.float32),
                  jax.ShapeDtypeStruct((B,), jnp.float32)),
        mesh=mesh,
        compiler_params=pltpu.CompilerParams(needs_layout_passes=False),
        scratch_types=[
            pltpu.VMEM((NCH, C), jnp.int32),
            pltpu.VMEM((NCH, C), jnp.int32),
            pltpu.VMEM((NCH, C), jnp.int32),
            pltpu.VMEM((3, C, D), jnp.float32),
            pltpu.VMEM((BPW,), jnp.float32),
            pltpu.VMEM((BPW,), jnp.float32),
            pltpu.SemaphoreType.DMA((3,)),
        ],
    )
    return f(user_inputs, pos_item_inputs, neg_item_inputs, user_table, item_table)


# speedup vs baseline: 1.0197x; 1.0197x over previous
import jax
import jax.numpy as jnp
from jax import lax
from jax.experimental import pallas as pl
from jax.experimental.pallas import tpu as pltpu
from jax.experimental.pallas import tpu_sc as plsc

B = 16384
D = 64
NC = 2
NS = 16
L = 16
NW = NC * NS
BPW = B // NW
C = 128
NCH = BPW // C


def _sc_body(u_idx, p_idx, n_idx, utab, itab, out_pos, out_neg,
             idx_u, idx_p, idx_n, rows_u, rows_p, rows_n,
             acc_pos_v, acc_neg_v, sems):
    wid = lax.axis_index("s") * NC + lax.axis_index("c")
    base = wid * BPW

    pltpu.sync_copy(u_idx.at[pl.ds(base, BPW)], idx_u)
    pltpu.sync_copy(p_idx.at[pl.ds(base, BPW)], idx_p)
    pltpu.sync_copy(n_idx.at[pl.ds(base, BPW)], idx_n)

    def issue(c, cb):
        @plsc.parallel_loop(0, C // L)
        def _issue(g):
            sl = pl.ds(c * C + g * L, L)
            vu = idx_u[sl]
            vp = idx_p[sl]
            vn = idx_n[sl]
            for j in range(L):
                d = pl.ds(g * L + j, 1)
                pltpu.async_copy(utab.at[pl.ds(vu[j], 1)], rows_u.at[cb, d],
                                 sems.at[cb, 0])
                pltpu.async_copy(itab.at[pl.ds(vp[j], 1)], rows_p.at[cb, d],
                                 sems.at[cb, 1])
                pltpu.async_copy(itab.at[pl.ds(vn[j], 1)], rows_n.at[cb, d],
                                 sems.at[cb, 2])

    def drain(cb):
        pltpu.make_async_copy(utab.at[pl.ds(0, C)], rows_u.at[cb],
                              sems.at[cb, 0]).wait()
        pltpu.make_async_copy(itab.at[pl.ds(0, C)], rows_p.at[cb],
                              sems.at[cb, 1]).wait()
        pltpu.make_async_copy(itab.at[pl.ds(0, C)], rows_n.at[cb],
                              sems.at[cb, 2]).wait()

    def compute(c, cb):
        for g in range(C // L):
            ev = lax.iota(jnp.int32, L) + g * L

            def col_body(_, carry):
                accp, accn, colv = carry
                vu = plsc.load_gather(rows_u.at[cb], [ev, colv])
                vp = plsc.load_gather(rows_p.at[cb], [ev, colv])
                vn = plsc.load_gather(rows_n.at[cb], [ev, colv])
                return (accp + vu * vp, accn + vu * vn, colv + 1)

            accp, accn, _ = lax.fori_loop(
                0, D, col_body,
                (jnp.zeros((L,), jnp.float32), jnp.zeros((L,), jnp.float32),
                 jnp.zeros((L,), jnp.int32)))
            acc_pos_v[pl.ds(c * C + g * L, L)] = accp
            acc_neg_v[pl.ds(c * C + g * L, L)] = accn

    issue(0, 0)
    if NCH > 1:
        issue(1, 1)
    for c in range(NCH):
        cb = c % 2
        drain(cb)
        compute(c, cb)
        if c + 2 < NCH:
            issue(c + 2, cb)

    pltpu.sync_copy(acc_pos_v, out_pos.at[pl.ds(base, BPW)])
    pltpu.sync_copy(acc_neg_v, out_neg.at[pl.ds(base, BPW)])


@jax.jit
def kernel(user_inputs, pos_item_inputs, neg_item_inputs, user_table, item_table):
    mesh = plsc.VectorSubcoreMesh(core_axis_name="c", subcore_axis_name="s")
    f = pl.kernel(
        _sc_body,
        out_type=(jax.ShapeDtypeStruct((B,), jnp.float32),
                  jax.ShapeDtypeStruct((B,), jnp.float32)),
        mesh=mesh,
        compiler_params=pltpu.CompilerParams(needs_layout_passes=False),
        scratch_types=[
            pltpu.VMEM((BPW,), jnp.int32),
            pltpu.VMEM((BPW,), jnp.int32),
            pltpu.VMEM((BPW,), jnp.int32),
            pltpu.VMEM((2, C, D), jnp.float32),
            pltpu.VMEM((2, C, D), jnp.float32),
            pltpu.VMEM((2, C, D), jnp.float32),
            pltpu.VMEM((BPW,), jnp.float32),
            pltpu.VMEM((BPW,), jnp.float32),
            pltpu.SemaphoreType.DMA((2, 3)),
        ],
    )
    return f(user_inputs, pos_item_inputs, neg_item_inputs, user_table, item_table)
